# baseline (speedup 1.0000x reference)
import functools

import jax
import jax.numpy as jnp
from jax import lax
from jax.experimental import pallas as pl
from jax.experimental.pallas import tpu as pltpu
from jax.experimental.pallas import tpu_sc as plsc

_ROWS = 128
_W = 16
_RIGHT0 = 16384
_NV = 1024.0
_NWORKERS = 32
_RPW = _ROWS // _NWORKERS


_SLAB = 128


def _mix_body(x_ref, a_v, b_v):
    wid = lax.axis_index("s") * 2 + lax.axis_index("c")
    r0 = wid * _RPW
    pltpu.sync_copy(x_ref.at[pl.ds(r0, _RPW), pl.ds(0, _SLAB)], a_v)
    pltpu.sync_copy(x_ref.at[pl.ds(r0, _RPW), pl.ds(_RIGHT0, _SLAB)], b_v)
    for i in range(_RPW):
        a = a_v[i, pl.ds(0, _W)]
        b = b_v[i, pl.ds(0, _W)]
        a_v[i, pl.ds(0, _W)] = jnp.mod(a + b, _NV) + 1.0
        b_v[i, pl.ds(0, _W)] = jnp.mod(_NV + a - b, _NV) + 1.0
    pltpu.sync_copy(a_v, x_ref.at[pl.ds(r0, _RPW), pl.ds(0, _SLAB)])
    pltpu.sync_copy(b_v, x_ref.at[pl.ds(r0, _RPW), pl.ds(_RIGHT0, _SLAB)])


_mix_fix = functools.partial(
    pl.kernel,
    mesh=plsc.VectorSubcoreMesh(core_axis_name="c", subcore_axis_name="s"),
    scratch_types=[
        pltpu.VMEM((_RPW, _SLAB), jnp.float32),
        pltpu.VMEM((_RPW, _SLAB), jnp.float32),
    ],
)(_mix_body)


def _copy_body(x_ref, y0, y1, y2, s0, s1, s2):
    c0 = pltpu.make_async_copy(x_ref.at[0], y0, s0)
    c1 = pltpu.make_async_copy(x_ref.at[1], y1, s1)
    c2 = pltpu.make_async_copy(x_ref.at[2], y2, s2)
    c0.start()
    c1.start()
    c2.start()
    c0.wait()
    c1.wait()
    c2.wait()


_bulk = pl.pallas_call(
    _copy_body,
    out_shape=(jax.ShapeDtypeStruct((_ROWS, 2 * _RIGHT0), jnp.float32),) * 3,
    in_specs=[pl.BlockSpec(memory_space=pl.ANY)],
    out_specs=(pl.BlockSpec(memory_space=pl.ANY),) * 3,
    scratch_shapes=[pltpu.SemaphoreType.DMA] * 3,
)


def kernel(x):
    y0, y1, y2 = _bulk(x)
    ref = jax.new_ref(y0)
    _mix_fix(ref)
    return (ref[...], y1, y2)


# speedup vs baseline: 29.5257x; 29.5257x over previous
import functools

import jax
import jax.numpy as jnp
from jax import lax
from jax.experimental import pallas as pl
from jax.experimental.pallas import tpu as pltpu
from jax.experimental.pallas import tpu_sc as plsc

_ROWS = 128
_W = 16
_RIGHT0 = 16384
_NV = 1024.0
_NWORKERS = 32
_RPW = _ROWS // _NWORKERS


_SLAB = 128


def _mix_body(x_ref, a_v, b_v):
    wid = lax.axis_index("s") * 2 + lax.axis_index("c")
    r0 = wid * _RPW
    pltpu.sync_copy(x_ref.at[pl.ds(r0, _RPW), pl.ds(0, _SLAB)], a_v)
    pltpu.sync_copy(x_ref.at[pl.ds(r0, _RPW), pl.ds(_RIGHT0, _SLAB)], b_v)
    for i in range(_RPW):
        a = a_v[i, pl.ds(0, _W)]
        b = b_v[i, pl.ds(0, _W)]
        a_v[i, pl.ds(0, _W)] = jnp.mod(a + b, _NV) + 1.0
        b_v[i, pl.ds(0, _W)] = jnp.mod(_NV + a - b, _NV) + 1.0
    pltpu.sync_copy(a_v, x_ref.at[pl.ds(r0, _RPW), pl.ds(0, _SLAB)])
    pltpu.sync_copy(b_v, x_ref.at[pl.ds(r0, _RPW), pl.ds(_RIGHT0, _SLAB)])


_mix_fix = functools.partial(
    pl.kernel,
    mesh=plsc.VectorSubcoreMesh(core_axis_name="c", subcore_axis_name="s"),
    scratch_types=[
        pltpu.VMEM((_RPW, _SLAB), jnp.float32),
        pltpu.VMEM((_RPW, _SLAB), jnp.float32),
    ],
)(_mix_body)


_COLS = 2 * _RIGHT0
_CW = 2048
_NCHUNK = _COLS // _CW


def _copy3_body(i0, i1, i2, o0, o1, o2):
    o0[...] = i0[0]
    o1[...] = i1[0]
    o2[...] = i2[0]


_bulk = pl.pallas_call(
    _copy3_body,
    grid=(_NCHUNK,),
    in_specs=[
        pl.BlockSpec((1, _ROWS, _CW), lambda j, k=k: (k, 0, j))
        for k in range(3)
    ],
    out_specs=[pl.BlockSpec((_ROWS, _CW), lambda j: (0, j))] * 3,
    out_shape=(jax.ShapeDtypeStruct((_ROWS, _COLS), jnp.float32),) * 3,
)


def kernel(x):
    y0, y1, y2 = _bulk(x, x, x)
    ref = jax.new_ref(y0)
    _mix_fix(ref)
    return (ref[...], y1, y2)


# speedup vs baseline: 31.2503x; 1.0584x over previous
import functools

import jax
import jax.numpy as jnp
from jax import lax
from jax.experimental import pallas as pl
from jax.experimental.pallas import tpu as pltpu
from jax.experimental.pallas import tpu_sc as plsc

_ROWS = 128
_COLS = 32768
_W = 16
_RIGHT0 = 16384
_NV = 1024.0
_SLAB = 128
_NWORKERS = 32
_RPW = _ROWS // _NWORKERS
_CW = 8192
_NCH = _COLS // _CW


def _mix_copy_body(x_ref, y_ref, a_v, b_v, c0, c1, si0, si1, so0, so1):
    wid = lax.axis_index("s") * 2 + lax.axis_index("c")
    rows = pl.ds(wid * _RPW, _RPW)
    pltpu.sync_copy(x_ref.at[0, rows, pl.ds(0, _SLAB)], a_v)
    pltpu.sync_copy(x_ref.at[0, rows, pl.ds(_RIGHT0, _SLAB)], b_v)
    for i in range(_RPW):
        a = a_v[i, pl.ds(0, _W)]
        b = b_v[i, pl.ds(0, _W)]
        a_v[i, pl.ds(0, _W)] = jnp.mod(a + b, _NV) + 1.0
        b_v[i, pl.ds(0, _W)] = jnp.mod(_NV + a - b, _NV) + 1.0

    bufs, isems, osems = (c0, c1), (si0, si1), (so0, so1)

    def in_copy(c):
        return pltpu.make_async_copy(
            x_ref.at[0, rows, pl.ds(c * _CW, _CW)], bufs[c % 2], isems[c % 2])

    def out_copy(c):
        return pltpu.make_async_copy(
            bufs[c % 2], y_ref.at[rows, pl.ds(c * _CW, _CW)], osems[c % 2])

    in_copy(0).start()
    in_copy(1).start()
    for c in range(_NCH):
        in_copy(c).wait()
        if c == 0:
            for i in range(_RPW):
                bufs[0][i, pl.ds(0, _W)] = a_v[i, pl.ds(0, _W)]
        if c == _RIGHT0 // _CW:
            for i in range(_RPW):
                bufs[c % 2][i, pl.ds(0, _W)] = b_v[i, pl.ds(0, _W)]
        out_copy(c).start()
        if c + 2 < _NCH:
            out_copy(c).wait()
            in_copy(c + 2).start()
    out_copy(_NCH - 2).wait()
    out_copy(_NCH - 1).wait()


_mix_copy = functools.partial(
    pl.kernel,
    out_type=jax.ShapeDtypeStruct((_ROWS, _COLS), jnp.float32),
    mesh=plsc.VectorSubcoreMesh(core_axis_name="c", subcore_axis_name="s"),
    scratch_types=[
        pltpu.VMEM((_RPW, _SLAB), jnp.float32),
        pltpu.VMEM((_RPW, _SLAB), jnp.float32),
        pltpu.VMEM((_RPW, _CW), jnp.float32),
        pltpu.VMEM((_RPW, _CW), jnp.float32),
        pltpu.SemaphoreType.DMA,
        pltpu.SemaphoreType.DMA,
        pltpu.SemaphoreType.DMA,
        pltpu.SemaphoreType.DMA,
    ],
)(_mix_copy_body)


def kernel(x):
    y0 = _mix_copy(x)
    return (y0, x[1], x[2])


# speedup vs baseline: 31.7940x; 1.0174x over previous
import functools

import jax
import jax.numpy as jnp
from jax import lax
from jax.experimental import pallas as pl
from jax.experimental.pallas import tpu as pltpu
from jax.experimental.pallas import tpu_sc as plsc

_ROWS = 128
_COLS = 32768
_W = 16
_RIGHT0 = 16384
_NV = 1024.0
_SLAB = 128
_NWORKERS = 32
_RPW = _ROWS // _NWORKERS
_CW = 4096
_NCH = _COLS // _CW
_NBUF = 4


def _mix_copy_body(x_ref, y_ref, a_v, b_v, *rest):
    bufs, isems, osems = rest[:_NBUF], rest[_NBUF:2 * _NBUF], rest[2 * _NBUF:]
    wid = lax.axis_index("s") * 2 + lax.axis_index("c")
    rows = pl.ds(wid * _RPW, _RPW)

    def in_copy(c):
        return pltpu.make_async_copy(
            x_ref.at[0, rows, pl.ds(c * _CW, _CW)],
            bufs[c % _NBUF], isems[c % _NBUF])

    def out_copy(c):
        return pltpu.make_async_copy(
            bufs[c % _NBUF], y_ref.at[rows, pl.ds(c * _CW, _CW)],
            osems[c % _NBUF])

    for c in range(_NBUF):
        in_copy(c).start()
    pltpu.sync_copy(x_ref.at[0, rows, pl.ds(0, _SLAB)], a_v)
    pltpu.sync_copy(x_ref.at[0, rows, pl.ds(_RIGHT0, _SLAB)], b_v)
    for i in range(_RPW):
        a = a_v[i, pl.ds(0, _W)]
        b = b_v[i, pl.ds(0, _W)]
        a_v[i, pl.ds(0, _W)] = jnp.mod(a + b, _NV) + 1.0
        b_v[i, pl.ds(0, _W)] = jnp.mod(_NV + a - b, _NV) + 1.0

    for c in range(_NCH):
        in_copy(c).wait()
        if c == 0:
            for i in range(_RPW):
                bufs[0][i, pl.ds(0, _W)] = a_v[i, pl.ds(0, _W)]
        if c == _RIGHT0 // _CW:
            for i in range(_RPW):
                bufs[c % _NBUF][i, pl.ds(0, _W)] = b_v[i, pl.ds(0, _W)]
        out_copy(c).start()
        if c + _NBUF < _NCH:
            out_copy(c).wait()
            in_copy(c + _NBUF).start()
    for c in range(_NCH - _NBUF, _NCH):
        out_copy(c).wait()


_mix_copy = functools.partial(
    pl.kernel,
    out_type=jax.ShapeDtypeStruct((_ROWS, _COLS), jnp.float32),
    mesh=plsc.VectorSubcoreMesh(core_axis_name="c", subcore_axis_name="s"),
    scratch_types=(
        [pltpu.VMEM((_RPW, _SLAB), jnp.float32)] * 2
        + [pltpu.VMEM((_RPW, _CW), jnp.float32)] * _NBUF
        + [pltpu.SemaphoreType.DMA] * (2 * _NBUF)
    ),
)(_mix_copy_body)


def kernel(x):
    y0 = _mix_copy(x)
    return (y0, x[1], x[2])


# speedup vs baseline: 31.8470x; 1.0017x over previous
import functools

import jax
import jax.numpy as jnp
from jax import lax
from jax.experimental import pallas as pl
from jax.experimental.pallas import tpu as pltpu
from jax.experimental.pallas import tpu_sc as plsc

_ROWS = 128
_COLS = 32768
_W = 16
_RIGHT0 = 16384
_NV = 1024.0
_SLAB = 128
_NWORKERS = 32
_RPW = _ROWS // _NWORKERS
_NBUF = 3


def _mix_copy_body(x_ref, y_ref, a_v, b_v, *rest):
    bufs, isems, osems = rest[:_NBUF], rest[_NBUF:2 * _NBUF], rest[2 * _NBUF:]
    wid = lax.axis_index("s") * 2 + lax.axis_index("c")
    r0 = wid * _RPW
    rows = pl.ds(r0, _RPW)

    def in_copy(c):
        return pltpu.make_async_copy(
            x_ref.at[0, pl.ds(r0 + c, 1), :], bufs[c % _NBUF],
            isems[c % _NBUF])

    def out_copy(c):
        return pltpu.make_async_copy(
            bufs[c % _NBUF], y_ref.at[pl.ds(r0 + c, 1), :],
            osems[c % _NBUF])

    for c in range(_NBUF):
        in_copy(c).start()
    pltpu.sync_copy(x_ref.at[0, rows, pl.ds(0, _SLAB)], a_v)
    pltpu.sync_copy(x_ref.at[0, rows, pl.ds(_RIGHT0, _SLAB)], b_v)
    for i in range(_RPW):
        a = a_v[i, pl.ds(0, _W)]
        b = b_v[i, pl.ds(0, _W)]
        a_v[i, pl.ds(0, _W)] = jnp.mod(a + b, _NV) + 1.0
        b_v[i, pl.ds(0, _W)] = jnp.mod(_NV + a - b, _NV) + 1.0

    for c in range(_RPW):
        in_copy(c).wait()
        buf = bufs[c % _NBUF]
        buf[0, pl.ds(0, _W)] = a_v[c, pl.ds(0, _W)]
        buf[0, pl.ds(_RIGHT0, _W)] = b_v[c, pl.ds(0, _W)]
        out_copy(c).start()
        if c + _NBUF < _RPW:
            out_copy(c).wait()
            in_copy(c + _NBUF).start()
    for c in range(max(0, _RPW - _NBUF), _RPW):
        out_copy(c).wait()


_mix_copy = functools.partial(
    pl.kernel,
    out_type=jax.ShapeDtypeStruct((_ROWS, _COLS), jnp.float32),
    mesh=plsc.VectorSubcoreMesh(core_axis_name="c", subcore_axis_name="s"),
    scratch_types=(
        [pltpu.VMEM((_RPW, _SLAB), jnp.float32)] * 2
        + [pltpu.VMEM((1, _COLS), jnp.float32)] * _NBUF
        + [pltpu.SemaphoreType.DMA] * (2 * _NBUF)
    ),
)(_mix_copy_body)


def kernel(x):
    y0 = _mix_copy(x)
    return (y0, x[1], x[2])
